# baseline (speedup 1.0000x reference)
import functools

import jax
import jax.numpy as jnp
from jax import lax
from jax.experimental import pallas as pl
from jax.experimental.pallas import tpu as pltpu
from jax.experimental.pallas import tpu_sc as plsc

DIM = 32
NUM_WORKERS = 32
BT = 128
L = 50
NBT = 128
BT_PER_W = NBT // NUM_WORKERS
BLOCKS_PER_W = BT_PER_W * L


def _emb_kernel():
    mesh = plsc.VectorSubcoreMesh(core_axis_name="c", subcore_axis_name="s")

    @functools.partial(
        pl.kernel,
        mesh=mesh,
        out_type=jax.ShapeDtypeStruct((L, 4, NBT, 8, BT), jnp.float32),
        scratch_types=[
            pltpu.VMEM((BT_PER_W, L, BT), jnp.int32),
            pltpu.VMEM((BT, DIM), jnp.float32),
            pltpu.VMEM((BT, DIM), jnp.float32),
            pltpu.VMEM((DIM, BT), jnp.float32),
            pltpu.VMEM((DIM, BT), jnp.float32),
            pltpu.SemaphoreType.DMA,
            pltpu.SemaphoreType.DMA,
        ],
        compiler_params=pltpu.CompilerParams(
            use_tc_tiling_on_sc=False, needs_layout_passes=False
        ),
    )
    def k(table, idxq, out5, idx_v, rows0, rows1, tr0, tr1, gsem, wsem):
        c = lax.axis_index("c")
        s = lax.axis_index("s")
        wid = s * 2 + c
        rows = [rows0, rows1]
        trs = [tr0, tr1]

        pltpu.sync_copy(idxq.at[pl.ds(wid * BT_PER_W, BT_PER_W)], idx_v)

        iota16 = lax.iota(jnp.int32, 16)
        rvecs = [iota16 + (ch * 16) for ch in range(8)]

        def issue_gather(b, buf):
            btk = b // L
            li = lax.rem(b, L)
            return pltpu.async_copy(table.at[idx_v.at[btk, li]], rows[buf], gsem)

        def do_block(si, par):
            b = si * 2 + par
            btk = b // L
            li = lax.rem(b, L)
            pltpu.make_async_copy(
                table.at[idx_v.at[btk, li]], rows[par], gsem
            ).wait()
            if par == 0:
                issue_gather(b + 1, 1 - par)
            else:
                @pl.when(si < BLOCKS_PER_W // 2 - 1)
                def _():
                    issue_gather(b + 1, 1 - par)

            rowsb = rows[par]
            trb = trs[par]

            @pl.when(si >= 1)
            def _():
                for dt in range(4):
                    pltpu.make_async_copy(
                        trb.at[pl.ds(dt * 8, 8)], out5.at[li, dt, 0], wsem
                    ).wait()

            for cc in range(DIM):
                col = jnp.full((16,), cc, dtype=jnp.int32)
                for ch in range(8):
                    v = plsc.load_gather(rowsb, [rvecs[ch], col])
                    trb[cc, pl.ds(ch * 16, 16)] = v

            bt_abs = wid * BT_PER_W + btk
            for dt in range(4):
                pltpu.async_copy(
                    trb.at[pl.ds(dt * 8, 8)], out5.at[li, dt, bt_abs], wsem
                )

        issue_gather(0, 0)

        def loop_body(si, carry):
            do_block(si, 0)
            do_block(si, 1)
            return carry

        lax.fori_loop(0, BLOCKS_PER_W // 2, loop_body, 0)

        for par in range(2):
            for dt in range(4):
                pltpu.make_async_copy(
                    trs[par].at[pl.ds(dt * 8, 8)], out5.at[0, dt, 0], wsem
                ).wait()

    return k


def kernel(input_, table):
    B, _ = input_.shape
    idxq = (
        input_.astype(jnp.int32)
        .T.reshape(L, NBT, BT)
        .transpose(1, 0, 2)
    )
    out5 = _emb_kernel()(table, idxq)
    return out5.transpose(2, 4, 0, 1, 3).reshape(B, L, DIM)


# speedup vs baseline: 1.0643x; 1.0643x over previous
import functools

import jax
import jax.numpy as jnp
from jax import lax
from jax.experimental import pallas as pl
from jax.experimental.pallas import tpu as pltpu
from jax.experimental.pallas import tpu_sc as plsc

DIM = 32
NUM_WORKERS = 32
BT = 128
L = 50
NBT = 128
BT_PER_W = NBT // NUM_WORKERS
RPW = BT_PER_W * BT


def _emb_kernel():
    mesh = plsc.VectorSubcoreMesh(core_axis_name="c", subcore_axis_name="s")

    @functools.partial(
        pl.kernel,
        mesh=mesh,
        out_type=jax.ShapeDtypeStruct((L, 4, NBT, 8, BT), jnp.float32),
        scratch_types=[
            pltpu.VMEM((L, RPW), jnp.int32),
            pltpu.VMEM((RPW, DIM), jnp.float32),
            pltpu.VMEM((RPW, DIM), jnp.float32),
            pltpu.VMEM((4, BT_PER_W, 8, BT), jnp.float32),
            pltpu.VMEM((4, BT_PER_W, 8, BT), jnp.float32),
            pltpu.SemaphoreType.DMA,
            pltpu.SemaphoreType.DMA,
        ],
        compiler_params=pltpu.CompilerParams(
            use_tc_tiling_on_sc=False, needs_layout_passes=False
        ),
    )
    def k(table, idxq, out5, idx_v, rows0, rows1, tr0, tr1, gsem, wsem):
        c = lax.axis_index("c")
        s = lax.axis_index("s")
        wid = s * 2 + c
        rows = [rows0, rows1]
        trs = [tr0, tr1]

        pltpu.sync_copy(idxq.at[wid], idx_v)

        iota16 = lax.iota(jnp.int32, 16)

        def issue_gather(li, buf):
            return pltpu.async_copy(table.at[idx_v.at[li]], rows[buf], gsem)

        def do_pos(si, par):
            li = si * 2 + par
            rowsb = rows[par]
            trb = trs[par]
            pltpu.make_async_copy(table.at[idx_v.at[li]], rowsb, gsem).wait()
            if par == 0:
                issue_gather(li + 1, 1 - par)
            else:
                @pl.when(si < L // 2 - 1)
                def _():
                    issue_gather(li + 1, 1 - par)

            @pl.when(si >= 1)
            def _():
                for dt in range(4):
                    pltpu.make_async_copy(
                        trb.at[dt], out5.at[0, dt, pl.ds(0, BT_PER_W)], wsem
                    ).wait()

            for dt in range(4):
                def tbody(di, carry, dt=dt):
                    col = jnp.broadcast_to(di + 8 * dt, (16,)).astype(jnp.int32)
                    for btk in range(BT_PER_W):
                        for ch in range(8):
                            rvec = iota16 + (btk * BT + ch * 16)
                            v = plsc.load_gather(rowsb, [rvec, col])
                            trb[dt, btk, di, pl.ds(ch * 16, 16)] = v
                    return carry

                lax.fori_loop(0, 8, tbody, 0)

            for dt in range(4):
                pltpu.async_copy(
                    trb.at[dt],
                    out5.at[li, dt, pl.ds(wid * BT_PER_W, BT_PER_W)],
                    wsem,
                )

        issue_gather(0, 0)

        def loop_body(si, carry):
            do_pos(si, 0)
            do_pos(si, 1)
            return carry

        lax.fori_loop(0, L // 2, loop_body, 0)

        for par in range(2):
            for dt in range(4):
                pltpu.make_async_copy(
                    trs[par].at[dt], out5.at[0, dt, pl.ds(0, BT_PER_W)], wsem
                ).wait()

    return k


def kernel(input_, table):
    B, _ = input_.shape
    idxq = (
        input_.astype(jnp.int32)
        .T.reshape(L, NUM_WORKERS, RPW)
        .transpose(1, 0, 2)
    )
    out5 = _emb_kernel()(table, idxq)
    return out5.transpose(2, 4, 0, 1, 3).reshape(B, L, DIM)


# speedup vs baseline: 1.4945x; 1.4042x over previous
import functools

import jax
import jax.numpy as jnp
from jax import lax
from jax.experimental import pallas as pl
from jax.experimental.pallas import tpu as pltpu
from jax.experimental.pallas import tpu_sc as plsc

DIM = 32
NUM_WORKERS = 32
BT = 128
L = 50
NBT = 128
BT_PER_W = NBT // NUM_WORKERS
RPW = BT_PER_W * BT
PADI = RPW + 1


def _emb_kernel():
    mesh = plsc.VectorSubcoreMesh(core_axis_name="c", subcore_axis_name="s")

    @functools.partial(
        pl.kernel,
        mesh=mesh,
        out_type=jax.ShapeDtypeStruct((L, 4, NBT, 8, BT), jnp.float32),
        scratch_types=[
            pltpu.VMEM((L, RPW), jnp.int32),
            pltpu.VMEM((RPW, DIM), jnp.float32),
            pltpu.VMEM((RPW, DIM), jnp.float32),
            pltpu.VMEM((DIM, PADI), jnp.float32),
            pltpu.VMEM((DIM, PADI), jnp.float32),
            pltpu.VMEM((4, BT_PER_W, 8, BT), jnp.float32),
            pltpu.VMEM((4, BT_PER_W, 8, BT), jnp.float32),
            pltpu.SemaphoreType.DMA,
            pltpu.SemaphoreType.DMA,
        ],
        compiler_params=pltpu.CompilerParams(
            use_tc_tiling_on_sc=False, needs_layout_passes=False
        ),
    )
    def k(table, idxq, out5, idx_v, rows0, rows1, tp0, tp1, tr0, tr1,
          gsem, wsem):
        c = lax.axis_index("c")
        s = lax.axis_index("s")
        wid = s * 2 + c
        rows = [rows0, rows1]
        tps = [tp0, tp1]
        trs = [tr0, tr1]

        pltpu.sync_copy(idxq.at[wid], idx_v)

        iota16 = lax.iota(jnp.int32, 16)
        cvec0 = iota16
        cvec1 = iota16 + 16

        def issue_gather(li, buf):
            return pltpu.async_copy(table.at[idx_v.at[li]], rows[buf], gsem)

        def do_pos(si, par):
            li = si * 2 + par
            rowsb = rows[par]
            tpb = tps[par]
            trb = trs[par]
            pltpu.make_async_copy(table.at[idx_v.at[li]], rowsb, gsem).wait()
            if par == 0:
                issue_gather(li + 1, 1 - par)
            else:
                @pl.when(si < L // 2 - 1)
                def _():
                    issue_gather(li + 1, 1 - par)

            @pl.when(si >= 1)
            def _():
                for dt in range(4):
                    pltpu.make_async_copy(
                        trb.at[dt], out5.at[0, dt, pl.ds(0, BT_PER_W)], wsem
                    ).wait()

            def abody(r, carry):
                bvec = jnp.broadcast_to(r, (16,)).astype(jnp.int32)
                v0 = rowsb[r, pl.ds(0, 16)]
                v1 = rowsb[r, pl.ds(16, 16)]
                plsc.store_scatter(tpb, [cvec0, bvec], v0)
                plsc.store_scatter(tpb, [cvec1, bvec], v1)
                return carry

            lax.fori_loop(0, RPW, abody, 0)

            def bbody(cc, carry):
                dt = cc // 8
                di = lax.rem(cc, 8)
                for btk in range(BT_PER_W):
                    for ch in range(8):
                        v = tpb[cc, pl.ds(btk * BT + ch * 16, 16)]
                        trb[dt, btk, di, pl.ds(ch * 16, 16)] = v
                return carry

            lax.fori_loop(0, DIM, bbody, 0)

            for dt in range(4):
                pltpu.async_copy(
                    trb.at[dt],
                    out5.at[li, dt, pl.ds(wid * BT_PER_W, BT_PER_W)],
                    wsem,
                )

        issue_gather(0, 0)

        def loop_body(si, carry):
            do_pos(si, 0)
            do_pos(si, 1)
            return carry

        lax.fori_loop(0, L // 2, loop_body, 0)

        for par in range(2):
            for dt in range(4):
                pltpu.make_async_copy(
                    trs[par].at[dt], out5.at[0, dt, pl.ds(0, BT_PER_W)], wsem
                ).wait()

    return k


def kernel(input_, table):
    B, _ = input_.shape
    idxq = (
        input_.astype(jnp.int32)
        .T.reshape(L, NUM_WORKERS, RPW)
        .transpose(1, 0, 2)
    )
    out5 = _emb_kernel()(table, idxq)
    return out5.transpose(2, 4, 0, 1, 3).reshape(B, L, DIM)


# speedup vs baseline: 1.5015x; 1.0047x over previous
import functools

import jax
import jax.numpy as jnp
from jax import lax
from jax.experimental import pallas as pl
from jax.experimental.pallas import tpu as pltpu
from jax.experimental.pallas import tpu_sc as plsc

DIM = 32
NUM_WORKERS = 32
BT = 128
L = 50
NBT = 128
BT_PER_W = NBT // NUM_WORKERS
RPW = BT_PER_W * BT
PADI = RPW + 1


def _emb_kernel():
    mesh = plsc.VectorSubcoreMesh(core_axis_name="c", subcore_axis_name="s")

    @functools.partial(
        pl.kernel,
        mesh=mesh,
        out_type=jax.ShapeDtypeStruct((L, 4, NBT, 8, BT), jnp.float32),
        scratch_types=[
            pltpu.VMEM((L, RPW), jnp.int32),
            pltpu.VMEM((RPW, DIM), jnp.float32),
            pltpu.VMEM((RPW, DIM), jnp.float32),
            pltpu.VMEM((DIM, PADI), jnp.float32),
            pltpu.VMEM((DIM, PADI), jnp.float32),
            pltpu.VMEM((4, BT_PER_W, 8, BT), jnp.float32),
            pltpu.VMEM((4, BT_PER_W, 8, BT), jnp.float32),
            pltpu.SemaphoreType.DMA,
            pltpu.SemaphoreType.DMA,
        ],
        compiler_params=pltpu.CompilerParams(
            use_tc_tiling_on_sc=False, needs_layout_passes=False
        ),
    )
    def k(table, idxq, out5, idx_v, rows0, rows1, tp0, tp1, tr0, tr1,
          gsem, wsem):
        c = lax.axis_index("c")
        s = lax.axis_index("s")
        wid = s * 2 + c
        rows = [rows0, rows1]
        tps = [tp0, tp1]
        trs = [tr0, tr1]

        pltpu.sync_copy(idxq.at[wid], idx_v)

        iota16 = lax.iota(jnp.int32, 16)
        cvec0 = iota16
        cvec1 = iota16 + 16

        def issue_gather(li, buf):
            return pltpu.async_copy(table.at[idx_v.at[li]], rows[buf], gsem)

        def do_pos(si, par):
            li = si * 2 + par
            rowsb = rows[par]
            tpb = tps[par]
            trb = trs[par]
            pltpu.make_async_copy(table.at[idx_v.at[li]], rowsb, gsem).wait()
            if par == 0:
                issue_gather(li + 1, 1 - par)
            else:
                @pl.when(si < L // 2 - 1)
                def _():
                    issue_gather(li + 1, 1 - par)

            @pl.when(si >= 1)
            def _():
                for dt in range(4):
                    pltpu.make_async_copy(
                        trb.at[dt], out5.at[0, dt, pl.ds(0, BT_PER_W)], wsem
                    ).wait()

            def abody(rr, carry):
                r0 = rr * 4
                for u in range(4):
                    r = r0 + u
                    bvec = jnp.broadcast_to(r, (16,)).astype(jnp.int32)
                    v0 = rowsb[r, pl.ds(0, 16)]
                    v1 = rowsb[r, pl.ds(16, 16)]
                    plsc.store_scatter(tpb, [cvec0, bvec], v0)
                    plsc.store_scatter(tpb, [cvec1, bvec], v1)
                return carry

            lax.fori_loop(0, RPW // 4, abody, 0)

            def bbody(cc, carry):
                dt = cc // 8
                di = lax.rem(cc, 8)
                for btk in range(BT_PER_W):
                    for ch in range(8):
                        v = tpb[cc, pl.ds(btk * BT + ch * 16, 16)]
                        trb[dt, btk, di, pl.ds(ch * 16, 16)] = v
                return carry

            lax.fori_loop(0, DIM, bbody, 0)

            for dt in range(4):
                pltpu.async_copy(
                    trb.at[dt],
                    out5.at[li, dt, pl.ds(wid * BT_PER_W, BT_PER_W)],
                    wsem,
                )

        issue_gather(0, 0)

        def loop_body(si, carry):
            do_pos(si, 0)
            do_pos(si, 1)
            return carry

        lax.fori_loop(0, L // 2, loop_body, 0)

        for par in range(2):
            for dt in range(4):
                pltpu.make_async_copy(
                    trs[par].at[dt], out5.at[0, dt, pl.ds(0, BT_PER_W)], wsem
                ).wait()

    return k


def kernel(input_, table):
    B, _ = input_.shape
    idxq = (
        input_.astype(jnp.int32)
        .T.reshape(L, NUM_WORKERS, RPW)
        .transpose(1, 0, 2)
    )
    out5 = _emb_kernel()(table, idxq)
    return out5.transpose(2, 4, 0, 1, 3).reshape(B, L, DIM)


# speedup vs baseline: 1.8565x; 1.2365x over previous
import functools

import jax
import jax.numpy as jnp
from jax import lax
from jax.experimental import pallas as pl
from jax.experimental.pallas import tpu as pltpu
from jax.experimental.pallas import tpu_sc as plsc

DIM = 32
NUM_WORKERS = 32
BT = 128
L = 50
NBT = 128
BT_PER_W = NBT // NUM_WORKERS
RPW = BT_PER_W * BT
PADI = RPW + 1


def _emb_kernel():
    mesh = plsc.VectorSubcoreMesh(core_axis_name="c", subcore_axis_name="s")

    @functools.partial(
        pl.kernel,
        mesh=mesh,
        out_type=jax.ShapeDtypeStruct((L, 4, NBT, 8, BT), jnp.float32),
        scratch_types=[
            pltpu.VMEM((L, RPW), jnp.int32),
            pltpu.VMEM((RPW, DIM), jnp.float32),
            pltpu.VMEM((RPW, DIM), jnp.float32),
            pltpu.VMEM((DIM, PADI), jnp.float32),
            pltpu.VMEM((DIM, PADI), jnp.float32),
            pltpu.SemaphoreType.DMA,
            pltpu.SemaphoreType.DMA,
        ],
        compiler_params=pltpu.CompilerParams(
            use_tc_tiling_on_sc=False, needs_layout_passes=False
        ),
    )
    def k(table, idxq, out5, idx_v, rows0, rows1, tp0, tp1, gsem, wsem):
        c = lax.axis_index("c")
        s = lax.axis_index("s")
        wid = s * 2 + c
        rows = [rows0, rows1]
        tps = [tp0, tp1]

        pltpu.sync_copy(idxq.at[wid], idx_v)

        iota16 = lax.iota(jnp.int32, 16)
        cvec0 = iota16
        cvec1 = iota16 + 16

        def issue_gather(li, buf):
            return pltpu.async_copy(table.at[idx_v.at[li]], rows[buf], gsem)

        def do_pos(si, par):
            li = si * 2 + par
            rowsb = rows[par]
            tpb = tps[par]
            pltpu.make_async_copy(table.at[idx_v.at[li]], rowsb, gsem).wait()
            if par == 0:
                issue_gather(li + 1, 1 - par)
            else:
                @pl.when(si < L // 2 - 1)
                def _():
                    issue_gather(li + 1, 1 - par)

            @pl.when(si >= 1)
            def _():
                for _i in range(4 * BT_PER_W):
                    pltpu.make_async_copy(
                        tpb.at[pl.ds(0, 8), pl.ds(0, BT)],
                        out5.at[0, 0, 0],
                        wsem,
                    ).wait()

            def abody(rr, carry):
                r0 = rr * 4
                for u in range(4):
                    r = r0 + u
                    bvec = jnp.broadcast_to(r, (16,)).astype(jnp.int32)
                    v0 = rowsb[r, pl.ds(0, 16)]
                    v1 = rowsb[r, pl.ds(16, 16)]
                    plsc.store_scatter(tpb, [cvec0, bvec], v0)
                    plsc.store_scatter(tpb, [cvec1, bvec], v1)
                return carry

            lax.fori_loop(0, RPW // 4, abody, 0)

            for dt in range(4):
                for btk in range(BT_PER_W):
                    pltpu.async_copy(
                        tpb.at[pl.ds(dt * 8, 8), pl.ds(btk * BT, BT)],
                        out5.at[li, dt, wid * BT_PER_W + btk],
                        wsem,
                    )

        issue_gather(0, 0)

        def loop_body(si, carry):
            do_pos(si, 0)
            do_pos(si, 1)
            return carry

        lax.fori_loop(0, L // 2, loop_body, 0)

        for _i in range(2 * 4 * BT_PER_W):
            pltpu.make_async_copy(
                tps[0].at[pl.ds(0, 8), pl.ds(0, BT)], out5.at[0, 0, 0], wsem
            ).wait()

    return k


def kernel(input_, table):
    B, _ = input_.shape
    idxq = (
        input_.astype(jnp.int32)
        .T.reshape(L, NUM_WORKERS, RPW)
        .transpose(1, 0, 2)
    )
    out5 = _emb_kernel()(table, idxq)
    return out5.transpose(2, 4, 0, 1, 3).reshape(B, L, DIM)
